# baseline (speedup 1.0000x reference)
import functools

import jax
import jax.numpy as jnp
from jax import lax
from jax.experimental import pallas as pl
from jax.experimental.pallas import tpu as pltpu
from jax.experimental.pallas import tpu_sc as plsc

B = 16384
K = 32
L = 16

_info = plsc.get_sparse_core_info()
NC = _info.num_cores
NS = _info.num_subcores
NW = NC * NS
BPW = B // NW
NG = BPW // L
CH = 128

_mesh = plsc.VectorSubcoreMesh(core_axis_name="c", subcore_axis_name="s")


@functools.partial(
    pl.kernel,
    mesh=_mesh,
    compiler_params=pltpu.CompilerParams(
        needs_layout_passes=False,
        use_tc_tiling_on_sc=False,
    ),
    out_type=jax.ShapeDtypeStruct((B,), jnp.float32),
    scratch_types=[
        pltpu.VMEM((BPW,), jnp.int32),
        pltpu.VMEM((BPW,), jnp.int32),
        pltpu.VMEM((BPW, K), jnp.float32),
        pltpu.VMEM((BPW, K), jnp.float32),
        pltpu.VMEM((BPW, 1), jnp.float32),
        pltpu.VMEM((BPW, 1), jnp.float32),
        pltpu.VMEM((L,), jnp.float32),
        pltpu.VMEM((BPW,), jnp.float32),
        pltpu.SemaphoreType.DMA,
        pltpu.SemaphoreType.DMA,
    ],
)
def _sc_predict(u_hbm, i_hbm, user_hbm, bu_hbm, movie_hbm, bm_hbm, mean_hbm,
                out_hbm, idx_u, idx_i, ue_v, me_v, bu_v, bm_v,
                mean_v, out_v, sem, bsem):
    wid = lax.axis_index("s") * NC + lax.axis_index("c")
    base = wid * BPW

    pltpu.sync_copy(u_hbm.at[pl.ds(base, BPW)], idx_u)
    pltpu.sync_copy(i_hbm.at[pl.ds(base, BPW)], idx_i)
    pltpu.sync_copy(mean_hbm, mean_v)

    copies = []
    for c in range(BPW // CH):
        s = pl.ds(c * CH, CH)
        copies.append(pltpu.async_copy(user_hbm.at[idx_u.at[s]], ue_v.at[s], sem))
        copies.append(pltpu.async_copy(movie_hbm.at[idx_i.at[s]], me_v.at[s], sem))

    def bias_body(g, _):
        s = pl.ds(pl.multiple_of(g * L, L), L)
        uvec = idx_u[s]
        ivec = idx_i[s]
        for l in range(L):
            j = g * L + l
            pltpu.async_copy(bu_hbm.at[pl.ds(uvec[l], 1), :],
                             bu_v.at[pl.ds(j, 1), :], bsem)
            pltpu.async_copy(bm_hbm.at[pl.ds(ivec[l], 1), :],
                             bm_v.at[pl.ds(j, 1), :], bsem)
        return 0

    lax.fori_loop(0, NG, bias_body, 0)

    for c in copies:
        c.wait()
    pltpu.make_async_copy(bu_hbm.at[pl.ds(0, BPW), :], bu_v, bsem).wait()
    pltpu.make_async_copy(bm_hbm.at[pl.ds(0, BPW), :], bm_v, bsem).wait()

    mean = mean_v[...]
    zeros = jnp.zeros((L,), jnp.int32)

    def group_body(g, _):
        s = pl.ds(pl.multiple_of(g * L, L), L)
        rid = g * L + lax.iota(jnp.int32, L)
        acc = jnp.zeros((L,), jnp.float32)
        for k in range(K):
            kk = jnp.full((L,), k, jnp.int32)
            uc = plsc.load_gather(ue_v, [rid, kk])
            mc = plsc.load_gather(me_v, [rid, kk])
            acc = acc + uc * mc
        bu = plsc.load_gather(bu_v, [rid, zeros])
        bm = plsc.load_gather(bm_v, [rid, zeros])
        out_v[s] = acc + bu + bm + mean
        return 0

    lax.fori_loop(0, NG, group_body, 0)

    pltpu.sync_copy(out_v, out_hbm.at[pl.ds(base, BPW)])


def kernel(u, i, user, bias_user, movie, bias_movie, mean):
    mean_v = jnp.full((L,), mean, dtype=jnp.float32)
    return _sc_predict(u, i, user, bias_user, movie, bias_movie, mean_v)


# speedup vs baseline: 2.5581x; 2.5581x over previous
import functools

import jax
import jax.numpy as jnp
from jax import lax
from jax.experimental import pallas as pl
from jax.experimental.pallas import tpu as pltpu
from jax.experimental.pallas import tpu_sc as plsc

B = 16384
K = 32
L = 16
W = 128

_info = plsc.get_sparse_core_info()
NC = _info.num_cores
NS = _info.num_subcores
NW = NC * NS
BPW = B // NW
NG = BPW // L
CH = 128
NR = BPW // CH

_mesh = plsc.VectorSubcoreMesh(core_axis_name="c", subcore_axis_name="s")


def _pad128(n):
    rows = -(-n // W)
    rows += (-rows) % 8
    return rows


@functools.partial(
    pl.kernel,
    mesh=_mesh,
    compiler_params=pltpu.CompilerParams(
        needs_layout_passes=False,
        use_tc_tiling_on_sc=False,
    ),
    out_type=jax.ShapeDtypeStruct((B,), jnp.float32),
    scratch_types=[
        pltpu.VMEM((BPW,), jnp.int32),
        pltpu.VMEM((BPW,), jnp.int32),
        pltpu.VMEM((BPW,), jnp.int32),
        pltpu.VMEM((BPW,), jnp.int32),
        pltpu.VMEM((BPW, K), jnp.float32),
        pltpu.VMEM((BPW, K), jnp.float32),
        pltpu.VMEM((CH, W), jnp.float32),
        pltpu.VMEM((CH, W), jnp.float32),
        pltpu.VMEM((BPW,), jnp.float32),
        pltpu.VMEM((BPW,), jnp.float32),
        pltpu.VMEM((L,), jnp.float32),
        pltpu.VMEM((BPW,), jnp.float32),
        pltpu.SemaphoreType.DMA,
        pltpu.SemaphoreType.DMA,
    ],
)
def _sc_predict(u_hbm, i_hbm, user_hbm, bu_hbm, movie_hbm, bm_hbm, mean_hbm,
                out_hbm, idx_u, idx_i, hi_u, hi_i, ue_v, me_v, brow_u, brow_m,
                bu_val, bm_val, mean_v, out_v, sem, bsem):
    wid = lax.axis_index("s") * NC + lax.axis_index("c")
    base = wid * BPW

    pltpu.sync_copy(u_hbm.at[pl.ds(base, BPW)], idx_u)
    pltpu.sync_copy(i_hbm.at[pl.ds(base, BPW)], idx_i)
    pltpu.sync_copy(mean_hbm, mean_v)

    copies = []
    for c in range(NR):
        s = pl.ds(c * CH, CH)
        copies.append(pltpu.async_copy(user_hbm.at[idx_u.at[s]], ue_v.at[s], sem))
        copies.append(pltpu.async_copy(movie_hbm.at[idx_i.at[s]], me_v.at[s], sem))

    def hi_body(c, _):
        s = pl.ds(pl.multiple_of(c * L, L), L)
        hi_u[s] = lax.shift_right_logical(idx_u[s], 7)
        hi_i[s] = lax.shift_right_logical(idx_i[s], 7)
        return 0

    lax.fori_loop(0, NG, hi_body, 0)

    lo_mask = jnp.full((L,), W - 1, jnp.int32)
    rid_g = lax.iota(jnp.int32, L)

    for r in range(NR):
        s = pl.ds(r * CH, CH)
        cu = pltpu.async_copy(bu_hbm.at[hi_u.at[s]], brow_u, bsem)
        cm = pltpu.async_copy(bm_hbm.at[hi_i.at[s]], brow_m, bsem)
        cu.wait()
        cm.wait()
        for g in range(CH // L):
            sg = pl.ds(r * CH + g * L, L)
            rid = g * L + rid_g
            bu_val[sg] = plsc.load_gather(brow_u, [rid, idx_u[sg] & lo_mask])
            bm_val[sg] = plsc.load_gather(brow_m, [rid, idx_i[sg] & lo_mask])

    for c in copies:
        c.wait()

    mean = mean_v[...]

    def group_body(g, _):
        s = pl.ds(pl.multiple_of(g * L, L), L)
        rid = g * L + lax.iota(jnp.int32, L)
        acc = jnp.zeros((L,), jnp.float32)
        for k in range(K):
            kk = jnp.full((L,), k, jnp.int32)
            uc = plsc.load_gather(ue_v, [rid, kk])
            mc = plsc.load_gather(me_v, [rid, kk])
            acc = acc + uc * mc
        out_v[s] = acc + bu_val[s] + bm_val[s] + mean
        return 0

    lax.fori_loop(0, NG, group_body, 0)

    pltpu.sync_copy(out_v, out_hbm.at[pl.ds(base, BPW)])


def kernel(u, i, user, bias_user, movie, bias_movie, mean):
    bu_rows = _pad128(bias_user.shape[0])
    bm_rows = _pad128(bias_movie.shape[0])
    bu128 = jnp.pad(bias_user.reshape(-1),
                    (0, bu_rows * W - bias_user.shape[0])).reshape(bu_rows, W)
    bm128 = jnp.pad(bias_movie.reshape(-1),
                    (0, bm_rows * W - bias_movie.shape[0])).reshape(bm_rows, W)
    mean_v = jnp.full((L,), mean, dtype=jnp.float32)
    return _sc_predict(u, i, user, bu128, movie, bm128, mean_v)


# speedup vs baseline: 2.5627x; 1.0018x over previous
import functools

import jax
import jax.numpy as jnp
from jax import lax
from jax.experimental import pallas as pl
from jax.experimental.pallas import tpu as pltpu
from jax.experimental.pallas import tpu_sc as plsc

B = 16384
K = 32
L = 16
W = 128

_info = plsc.get_sparse_core_info()
NC = _info.num_cores
NS = _info.num_subcores
NW = NC * NS
BPW = B // NW
NG = BPW // L
CH = 128
NR = BPW // CH

_mesh = plsc.VectorSubcoreMesh(core_axis_name="c", subcore_axis_name="s")


def _pad128(n):
    rows = -(-n // W)
    rows += (-rows) % 8
    return rows


@functools.partial(
    pl.kernel,
    mesh=_mesh,
    compiler_params=pltpu.CompilerParams(
        needs_layout_passes=False,
        use_tc_tiling_on_sc=False,
    ),
    out_type=jax.ShapeDtypeStruct((B,), jnp.float32),
    scratch_types=[
        pltpu.VMEM((BPW,), jnp.int32),
        pltpu.VMEM((BPW,), jnp.int32),
        pltpu.VMEM((BPW,), jnp.int32),
        pltpu.VMEM((BPW,), jnp.int32),
        pltpu.VMEM((BPW, K), jnp.float32),
        pltpu.VMEM((BPW, K), jnp.float32),
        pltpu.VMEM((CH, W), jnp.float32),
        pltpu.VMEM((CH, W), jnp.float32),
        pltpu.VMEM((BPW,), jnp.float32),
        pltpu.VMEM((BPW,), jnp.float32),
        pltpu.VMEM((L,), jnp.float32),
        pltpu.VMEM((BPW,), jnp.float32),
        pltpu.SemaphoreType.DMA,
        pltpu.SemaphoreType.DMA,
    ],
)
def _sc_predict(u_hbm, i_hbm, user_hbm, bu_hbm, movie_hbm, bm_hbm, mean_hbm,
                out_hbm, idx_u, idx_i, hi_u, hi_i, ue_v, me_v, brow_u, brow_m,
                bu_val, bm_val, mean_v, out_v, sem, bsem):
    wid = lax.axis_index("s") * NC + lax.axis_index("c")
    base = wid * BPW

    pltpu.sync_copy(u_hbm.at[pl.ds(base, BPW)], idx_u)
    pltpu.sync_copy(i_hbm.at[pl.ds(base, BPW)], idx_i)
    pltpu.sync_copy(mean_hbm, mean_v)

    copies = []
    for c in range(NR):
        s = pl.ds(c * CH, CH)
        copies.append(pltpu.async_copy(user_hbm.at[idx_u.at[s]], ue_v.at[s], sem))
        copies.append(pltpu.async_copy(movie_hbm.at[idx_i.at[s]], me_v.at[s], sem))

    def hi_body(c, _):
        s = pl.ds(pl.multiple_of(c * L, L), L)
        hi_u[s] = lax.shift_right_logical(idx_u[s], 7)
        hi_i[s] = lax.shift_right_logical(idx_i[s], 7)
        return 0

    lax.fori_loop(0, NG, hi_body, 0)

    lo_mask = jnp.full((L,), W - 1, jnp.int32)
    rid_g = lax.iota(jnp.int32, L)

    for r in range(NR):
        s = pl.ds(r * CH, CH)
        cu = pltpu.async_copy(bu_hbm.at[hi_u.at[s]], brow_u, bsem)
        cm = pltpu.async_copy(bm_hbm.at[hi_i.at[s]], brow_m, bsem)
        cu.wait()
        cm.wait()
        for g in range(CH // L):
            sg = pl.ds(r * CH + g * L, L)
            rid = g * L + rid_g
            bu_val[sg] = plsc.load_gather(brow_u, [rid, idx_u[sg] & lo_mask])
            bm_val[sg] = plsc.load_gather(brow_m, [rid, idx_i[sg] & lo_mask])

    for c in copies:
        c.wait()

    mean = mean_v[...]

    def group_body(g, _):
        s = pl.ds(pl.multiple_of(g * L, L), L)
        rid = g * L + lax.iota(jnp.int32, L)
        acc = jnp.zeros((L,), jnp.float32)
        for k in range(K):
            kk = jnp.full((L,), k, jnp.int32)
            uc = plsc.load_gather(ue_v, [rid, kk])
            mc = plsc.load_gather(me_v, [rid, kk])
            acc = acc + uc * mc
        out_v[s] = acc + bu_val[s] + bm_val[s] + mean
        return 0

    lax.fori_loop(0, NG, group_body, 0)

    pltpu.sync_copy(out_v, out_hbm.at[pl.ds(base, BPW)])


def kernel(u, i, user, bias_user, movie, bias_movie, mean):
    bu_rows = _pad128(bias_user.shape[0])
    bm_rows = _pad128(bias_movie.shape[0])
    one = (mean - mean) + jnp.float32(1.0)
    bu128 = jnp.pad(bias_user.reshape(-1) * one,
                    (0, bu_rows * W - bias_user.shape[0])).reshape(bu_rows, W)
    bm128 = jnp.pad(bias_movie.reshape(-1) * one,
                    (0, bm_rows * W - bias_movie.shape[0])).reshape(bm_rows, W)
    mean_v = jnp.full((L,), mean, dtype=jnp.float32)
    return _sc_predict(u, i, user, bu128, movie, bm128, mean_v)
